# baseline (speedup 1.0000x reference)
import functools

import jax
import jax.numpy as jnp
from jax import lax
from jax.experimental import pallas as pl
from jax.experimental.pallas import tpu as pltpu
from jax.experimental.pallas import tpu_sc as plsc

NC = 2
NS = 16
NW = NC * NS
BK = 16


def kernel(input_, weight):
    B, S = input_.shape
    V, D = weight.shape
    total = B * S
    per_w = total // NW
    n_batches = per_w // BK
    n_rows = per_w // 128

    idx = input_.reshape(NW, n_rows, 128).astype(jnp.int32)
    w8 = weight.reshape(V // 8, 8, D)

    mesh = plsc.VectorSubcoreMesh(
        core_axis_name="c", subcore_axis_name="s", num_cores=NC, num_subcores=NS
    )

    @functools.partial(
        pl.kernel,
        out_type=jax.ShapeDtypeStruct(
            (NW * n_batches, BK // 2, 2 * D), jnp.float32
        ),
        mesh=mesh,
        scratch_types=[
            pltpu.VMEM((n_rows, 128), jnp.int32),
            pltpu.VMEM((2, BK, 8, D), jnp.float32),
            pltpu.VMEM((2, BK // 2, 2 * D), jnp.float32),
            pltpu.SemaphoreType.DMA((2,)),
            pltpu.SemaphoreType.DMA((2,)),
        ],
        compiler_params=pltpu.CompilerParams(
            use_tc_tiling_on_sc=True, needs_layout_passes=False
        ),
    )
    def emb(idx_hbm, w_hbm, out_hbm, idx_v, ring, stage_v, dsem, wsem):
        wid = lax.axis_index("s") * NC + lax.axis_index("c")
        pltpu.sync_copy(idx_hbm.at[wid], idx_v)

        bpr = 128 // BK
        bsh = bpr.bit_length() - 1

        def batch_vecs(b, g):
            row = lax.shift_right_logical(b, bsh)
            col0 = (b & (bpr - 1)) * BK
            return idx_v[row, pl.ds(col0 + g * 16, 16)]

        def fire(b, h):
            for g in range(BK // 16):
                tv = lax.shift_right_logical(batch_vecs(b, g), 3)
                for l in range(16):
                    pltpu.async_copy(
                        w_hbm.at[tv[l]], ring.at[h, g * 16 + l], dsem.at[h]
                    )

        def drain(h):
            for j in range(BK):
                pltpu.make_async_copy(
                    w_hbm.at[0], ring.at[h, j], dsem.at[h]
                ).wait()

        def extract(b, h):
            for g in range(BK // 16):
                rv = batch_vecs(b, g) & 7
                for l in range(16):
                    r = rv[l]
                    j = g * 16 + l
                    j2, e = j // 2, j % 2
                    for p in range(D // 16):
                        stage_v[h, j2, pl.ds(e * D + p * 16, 16)] = (
                            ring[h, j, r, pl.ds(p * 16, 16)]
                        )

        def wait_write(b, h):
            pltpu.make_async_copy(
                stage_v.at[h], out_hbm.at[wid * n_batches + b], wsem.at[h]
            ).wait()

        fire(0, 0)

        @pl.loop(0, n_batches // 2)
        def body(jj):
            for bb in range(2):
                h = bb
                b = 2 * jj + bb

                @pl.when(b < n_batches - 1)
                def _():
                    fire(b + 1, 1 - h)

                drain(h)

                @pl.when(jj > 0)
                def _():
                    wait_write(b - 2, h)

                extract(b, h)
                pltpu.async_copy(
                    stage_v.at[h], out_hbm.at[wid * n_batches + b], wsem.at[h]
                )

        wait_write(n_batches - 2, 0)
        wait_write(n_batches - 1, 1)

    out = emb(idx, w8)
    return out.reshape(B, S, D)


# speedup vs baseline: 1.0506x; 1.0506x over previous
import functools

import jax
import jax.numpy as jnp
from jax import lax
from jax.experimental import pallas as pl
from jax.experimental.pallas import tpu as pltpu
from jax.experimental.pallas import tpu_sc as plsc

NC = 2
NS = 16
NW = NC * NS
BK = 32


def kernel(input_, weight):
    B, S = input_.shape
    V, D = weight.shape
    total = B * S
    per_w = total // NW
    n_batches = per_w // BK
    n_rows = per_w // 128

    idx = input_.reshape(NW, n_rows, 128).astype(jnp.int32)
    w8 = weight.reshape(V // 8, 8, D)

    mesh = plsc.VectorSubcoreMesh(
        core_axis_name="c", subcore_axis_name="s", num_cores=NC, num_subcores=NS
    )

    @functools.partial(
        pl.kernel,
        out_type=jax.ShapeDtypeStruct(
            (NW * n_batches, BK // 2, 2 * D), jnp.float32
        ),
        mesh=mesh,
        scratch_types=[
            pltpu.VMEM((n_rows, 128), jnp.int32),
            pltpu.VMEM((2, BK, 8, D), jnp.float32),
            pltpu.VMEM((2, BK // 2, 2 * D), jnp.float32),
            pltpu.SemaphoreType.DMA((2,)),
            pltpu.SemaphoreType.DMA((2,)),
        ],
        compiler_params=pltpu.CompilerParams(
            use_tc_tiling_on_sc=True, needs_layout_passes=False
        ),
    )
    def emb(idx_hbm, w_hbm, out_hbm, idx_v, ring, stage_v, dsem, wsem):
        wid = lax.axis_index("s") * NC + lax.axis_index("c")
        pltpu.sync_copy(idx_hbm.at[wid], idx_v)

        def batch_vecs(b, g):
            row = lax.shift_right_logical(b, 2)
            col0 = (b & 3) * BK
            return idx_v[row, pl.ds(col0 + g * 16, 16)]

        def fire(b, h):
            for g in range(2):
                tv = lax.shift_right_logical(batch_vecs(b, g), 3)
                for l in range(16):
                    pltpu.async_copy(
                        w_hbm.at[tv[l]], ring.at[h, g * 16 + l], dsem.at[h]
                    )

        def drain(h):
            for j in range(BK):
                pltpu.make_async_copy(
                    w_hbm.at[0], ring.at[h, j], dsem.at[h]
                ).wait()

        def extract(b, h):
            for g in range(2):
                rv = batch_vecs(b, g) & 7
                for l in range(16):
                    r = rv[l]
                    j = g * 16 + l
                    j2, e = j // 2, j % 2
                    for p in range(D // 16):
                        stage_v[h, j2, pl.ds(e * D + p * 16, 16)] = (
                            ring[h, j, r, pl.ds(p * 16, 16)]
                        )

        def wait_write(b, h):
            pltpu.make_async_copy(
                stage_v.at[h], out_hbm.at[wid * n_batches + b], wsem.at[h]
            ).wait()

        fire(0, 0)

        @pl.loop(0, n_batches // 2)
        def body(jj):
            for bb in range(2):
                h = bb
                b = 2 * jj + bb

                @pl.when(b < n_batches - 1)
                def _():
                    fire(b + 1, 1 - h)

                drain(h)

                @pl.when(jj > 0)
                def _():
                    wait_write(b - 2, h)

                extract(b, h)
                pltpu.async_copy(
                    stage_v.at[h], out_hbm.at[wid * n_batches + b], wsem.at[h]
                )

        wait_write(n_batches - 2, 0)
        wait_write(n_batches - 1, 1)

    out = emb(idx, w8)
    return out.reshape(B, S, D)
